# baseline (speedup 1.0000x reference)
import functools

import jax
import jax.numpy as jnp
from jax import lax
from jax.experimental import pallas as pl
from jax.experimental.pallas import tpu as pltpu
from jax.experimental.pallas import tpu_sc as plsc

N_V = 1723
NP = 1792
EPS = 1e-5


def _col_mask():
    it = lax.broadcasted_iota(jnp.int32, (1, NP), 1)
    return (it < N_V).astype(jnp.float32)


def _gn_relu(h, g, b, num_groups, mask):
    C = h.shape[0]
    gs = C // num_groups
    r0 = lax.broadcasted_iota(jnp.int32, (num_groups, C), 0)
    c0 = lax.broadcasted_iota(jnp.int32, (num_groups, C), 1)
    G = (c0 // gs == r0).astype(jnp.float32)
    r1 = lax.broadcasted_iota(jnp.int32, (C, num_groups), 0)
    c1 = lax.broadcasted_iota(jnp.int32, (C, num_groups), 1)
    GT = (r1 // gs == c1).astype(jnp.float32)
    cnt = float(gs * N_V)
    s1 = jnp.sum(h, axis=1, keepdims=True)
    s2 = jnp.sum(h * h, axis=1, keepdims=True)
    gm = jnp.dot(G, s1, preferred_element_type=jnp.float32) / cnt
    gm2 = jnp.dot(G, s2, preferred_element_type=jnp.float32) / cnt
    inv = lax.rsqrt(jnp.maximum(gm2 - gm * gm, 0.0) + EPS)
    mean_c = jnp.dot(GT, gm, preferred_element_type=jnp.float32)
    inv_c = jnp.dot(GT, inv, preferred_element_type=jnp.float32)
    scale = inv_c * g
    shift = b - mean_c * scale
    return jnp.maximum(h * scale + shift, 0.0) * mask


def _mm(W, x):
    return jnp.dot(W, x, preferred_element_type=jnp.float32)


def _mm_tl(W, x):
    return lax.dot_general(W, x, (((0,), (0,)), ((), ())),
                           preferred_element_type=jnp.float32)


def _gc0_body(x_ref, w_ref, b_ref, o_ref):
    j = pl.program_id(1)
    xb = x_ref[0]
    col = lax.broadcasted_iota(jnp.int32, xb.shape, 1) + j * xb.shape[1]
    xb = jnp.where(col < N_V, xb, 0.0)
    y = _mm(w_ref[...], xb) + b_ref[...]
    colo = lax.broadcasted_iota(jnp.int32, y.shape, 1) + j * y.shape[1]
    o_ref[0] = jnp.where(colo < N_V, y, 0.0)


def _gc0(x, W, bp):
    B, CK, _ = x.shape
    CO = W.shape[0]
    NB = 128
    return pl.pallas_call(
        _gc0_body,
        grid=(B, NP // NB),
        in_specs=[
            pl.BlockSpec((1, CK, NB), lambda i, j: (i, 0, j)),
            pl.BlockSpec((CO, CK), lambda i, j: (0, 0)),
            pl.BlockSpec((CO, 1), lambda i, j: (0, 0)),
        ],
        out_specs=pl.BlockSpec((1, CO, NB), lambda i, j: (i, 0, j)),
        out_shape=jax.ShapeDtypeStruct((B, CO, NP), jnp.float32),
    )(x, W, bp)


def _sl(pv, o, n):
    return lax.slice(pv, (o, 0), (o + n, 1))


def _block_vals(h, Bm, mats, pv, d, mask):
    in_c, half = d['in_c'], d['half']
    vo = d['vo']
    w1, cw, w2 = mats[0], mats[1], mats[2]
    y = _gn_relu(h, _sl(pv, vo['pre_norm_g'], in_c),
                 _sl(pv, vo['pre_norm_b'], in_c), in_c // 8, mask)
    y1 = (_mm(w1, y) + _sl(pv, vo['lin1_b'], half)) * mask
    y1 = _gn_relu(y1, _sl(pv, vo['norm1_g'], half),
                  _sl(pv, vo['norm1_b'], half), half // 8, mask)
    s_cm = _mm_tl(cw, y1)
    z = (jnp.dot(s_cm, Bm, preferred_element_type=jnp.float32)
         + _sl(pv, vo['conv_b'], half)) * mask
    y2 = _gn_relu(z, _sl(pv, vo['norm2_g'], half),
                  _sl(pv, vo['norm2_b'], half), half // 8, mask)
    y3 = _mm(w2, y2) + _sl(pv, vo['lin2_b'], d['out_c'])
    if d['skip']:
        hs = _mm(mats[3], h) + _sl(pv, vo['skip_b'], d['out_c'])
    else:
        hs = h
    return (hs + y3) * mask


def _mega_body(descs, nm, ho, *refs):
    h0_ref, B_ref, pv_ref = refs[0], refs[1], refs[2]
    mats = [r[...] for r in refs[3:3 + nm]]
    slw, slb, cgw, cgb, clw, clb, shp_ref, cam_ref = refs[3 + nm:]
    mask = _col_mask()
    pv = pv_ref[...]
    Bm = B_ref[...]

    h = h0_ref[0]
    for d in descs[:4]:
        nm_d = 4 if d['skip'] else 3
        h = _block_vals(h, Bm, mats[d['mi']:d['mi'] + nm_d], pv, d, mask)
    s = h
    for d in descs[4:]:
        nm_d = 4 if d['skip'] else 3
        s = _block_vals(s, Bm, mats[d['mi']:d['mi'] + nm_d], pv, d, mask)

    sn = _gn_relu(s, _sl(pv, ho['sg'], 32), _sl(pv, ho['sb'], 32), 4, mask)
    shp_ref[0] = _mm(slw[...], sn) + slb[...]
    hn = _gn_relu(h, _sl(pv, ho['cg'], 512), _sl(pv, ho['cb'], 512), 64, mask)
    c = jnp.maximum(_mm(cgw[...], hn) + cgb[...], 0.0) * mask
    cam_ref[0] = (jnp.dot(c, clw[...], preferred_element_type=jnp.float32)
                  + clb[...])


def _mega(h0, Bmat, params):
    B = h0.shape[0]
    blocks = list(params['gc_blocks']) + list(params['shape_blocks'])

    mats, vecs, descs = [], [], []
    voff = [0]

    def addvec(a):
        o = voff[0]
        vecs.append(a.reshape(-1, 1))
        voff[0] += a.shape[0]
        return o

    for p in blocks:
        d = {'in_c': p['lin1_W'].shape[1], 'half': p['conv_W'].shape[0],
             'out_c': p['lin2_W'].shape[0], 'skip': 'skip_W' in p,
             'mi': len(mats)}
        mats += [p['lin1_W'], p['conv_W'], p['lin2_W']]
        d['vo'] = {k: addvec(p[k]) for k in
                   ['pre_norm_g', 'pre_norm_b', 'lin1_b', 'norm1_g',
                    'norm1_b', 'conv_b', 'norm2_g', 'norm2_b', 'lin2_b']}
        if d['skip']:
            mats.append(p['skip_W'])
            d['vo']['skip_b'] = addvec(p['skip_b'])
        descs.append(d)

    ho = {'sg': addvec(params['shape_gn_g']),
          'sb': addvec(params['shape_gn_b']),
          'cg': addvec(params['cam_gn_g']),
          'cb': addvec(params['cam_gn_b'])}
    pv = jnp.concatenate(vecs, axis=0)

    slw = jnp.zeros((8, 32), jnp.float32).at[:3].set(params['shape_lin_W'])
    slb = jnp.zeros((8, 1), jnp.float32).at[:3, 0].set(params['shape_lin_b'])
    cgw = jnp.zeros((8, 512), jnp.float32).at[:1].set(params['cam_glin_W'])
    cgb = jnp.zeros((8, 1), jnp.float32).at[:1, 0].set(params['cam_glin_b'])
    clw = jnp.zeros((NP, 128), jnp.float32).at[:N_V, :3].set(
        params['cam_lin_W'].T)
    clb = jnp.zeros((1, 128), jnp.float32).at[0, :3].set(params['cam_lin_b'])

    ops = [h0, Bmat, pv] + mats + [slw, slb, cgw, cgb, clw, clb]
    specs = [pl.BlockSpec((1, h0.shape[1], NP), lambda i: (i, 0, 0))]
    specs += [pl.BlockSpec(a.shape, lambda i, _r=a.ndim: tuple(0 for _ in range(_r)))
              for a in ops[1:]]

    return pl.pallas_call(
        functools.partial(_mega_body, descs, len(mats), ho),
        grid=(B,),
        in_specs=specs,
        out_specs=[pl.BlockSpec((1, 8, NP), lambda i: (i, 0, 0)),
                   pl.BlockSpec((1, 8, 128), lambda i: (i, 0, 0))],
        out_shape=[jax.ShapeDtypeStruct((B, 8, NP), jnp.float32),
                   jax.ShapeDtypeStruct((B, 8, 128), jnp.float32)],
    )(*ops)


_HALF = 896
_EPT = 896
_NNZ_PAD = 16 * _EPT
_STRIPE = _HALF * NP // 16
_ZCH = _STRIPE // 8


def _sc_body(rows_hbm, cols_hbm, vals_hbm, out_hbm,
             sbuf, rowv, colv, valv, idx2, val2, zbuf, sem):
    cid = lax.axis_index("c")
    sid = lax.axis_index("s")
    base = sid * _EPT
    pltpu.sync_copy(rows_hbm.at[pl.ds(base, _EPT)], rowv)
    pltpu.sync_copy(cols_hbm.at[pl.ds(base, _EPT)], colv)
    pltpu.sync_copy(vals_hbm.at[pl.ds(base, _EPT)], valv)

    def zb(i, carry):
        zbuf[pl.ds(i * 16, 16)] = jnp.zeros((16,), jnp.float32)
        return carry
    lax.fori_loop(0, _ZCH // 16, zb, 0)
    for t in range(8):
        pltpu.sync_copy(zbuf, sbuf.at[pl.ds(sid * _STRIPE + t * _ZCH, _ZCH)])

    ofs = cid * _HALF
    for k in range(_EPT // 16):
        r16 = rowv[pl.ds(k * 16, 16)]
        c16 = colv[pl.ds(k * 16, 16)]
        inh = (c16 >= ofs) & (c16 < ofs + _HALF)
        flat = jnp.where(inh, (c16 - ofs) * NP + r16, N_V)
        idx2[k // 8, pl.ds((k % 8) * 16, 16)] = flat
        val2[k // 8, pl.ds((k % 8) * 16, 16)] = valv[pl.ds(k * 16, 16)]

    plsc.subcore_barrier()
    for j in range(7):
        pltpu.sync_copy(val2.at[j], sbuf.at[idx2.at[j]], add=True)
    plsc.subcore_barrier()
    r0 = cid * _HALF + sid * 56
    copies = []
    for r in range(56):
        copies.append(pltpu.async_copy(
            sbuf.at[pl.ds((sid * 56 + r) * NP, NP)],
            out_hbm.at[r0 + r], sem))
    for c in copies:
        c.wait()


def _build_B(adj_indices, adj_values):
    pad = _NNZ_PAD - adj_values.shape[0]
    rows = jnp.pad(adj_indices[0], (0, pad))
    cols = jnp.pad(adj_indices[1], (0, pad), constant_values=2 * NP)
    vals = jnp.pad(adj_values, (0, pad))

    mesh = plsc.VectorSubcoreMesh(core_axis_name="c", subcore_axis_name="s")
    run = functools.partial(
        pl.kernel,
        out_type=jax.ShapeDtypeStruct((NP, NP), jnp.float32),
        mesh=mesh,
        scratch_types=[
            pltpu.VMEM_SHARED((_HALF * NP,), jnp.float32),
            pltpu.VMEM((_EPT,), jnp.int32),
            pltpu.VMEM((_EPT,), jnp.int32),
            pltpu.VMEM((_EPT,), jnp.float32),
            pltpu.VMEM((7, 128), jnp.int32),
            pltpu.VMEM((7, 128), jnp.float32),
            pltpu.VMEM((_ZCH,), jnp.float32),
            pltpu.SemaphoreType.DMA,
        ],
    )(_sc_body)
    return run(rows, cols, vals)


def kernel(x, adj_indices, adj_values, params):
    Bmat = _build_B(adj_indices, adj_values)
    h = _gc0(x, params['gc0_W'], params['gc0_b'].reshape(-1, 1))
    shp, cam = _mega(h, Bmat, params)
    shape = shp[:, :3, :N_V]
    camera = cam[:, 0, :3]
    return (shape, camera)


# speedup vs baseline: 1.1142x; 1.1142x over previous
import functools

import jax
import jax.numpy as jnp
from jax import lax
from jax.experimental import pallas as pl
from jax.experimental.pallas import tpu as pltpu
from jax.experimental.pallas import tpu_sc as plsc

N_V = 1723
NP = 1792
EPS = 1e-5


def _col_mask():
    it = lax.broadcasted_iota(jnp.int32, (1, NP), 1)
    return (it < N_V).astype(jnp.float32)


def _gn_relu(h, g, b, num_groups, mask):
    C = h.shape[0]
    gs = C // num_groups
    r0 = lax.broadcasted_iota(jnp.int32, (num_groups, C), 0)
    c0 = lax.broadcasted_iota(jnp.int32, (num_groups, C), 1)
    G = (c0 // gs == r0).astype(jnp.float32)
    r1 = lax.broadcasted_iota(jnp.int32, (C, num_groups), 0)
    c1 = lax.broadcasted_iota(jnp.int32, (C, num_groups), 1)
    GT = (r1 // gs == c1).astype(jnp.float32)
    cnt = float(gs * N_V)
    s1 = jnp.sum(h, axis=1, keepdims=True)
    s2 = jnp.sum(h * h, axis=1, keepdims=True)
    gm = jnp.dot(G, s1, preferred_element_type=jnp.float32) / cnt
    gm2 = jnp.dot(G, s2, preferred_element_type=jnp.float32) / cnt
    inv = lax.rsqrt(jnp.maximum(gm2 - gm * gm, 0.0) + EPS)
    mean_c = jnp.dot(GT, gm, preferred_element_type=jnp.float32)
    inv_c = jnp.dot(GT, inv, preferred_element_type=jnp.float32)
    scale = inv_c * g
    shift = b - mean_c * scale
    return jnp.maximum(h * scale + shift, 0.0) * mask


def _mm(W, x):
    return jnp.dot(W, x, preferred_element_type=jnp.float32)


def _mm_tl(W, x):
    return lax.dot_general(W, x, (((0,), (0,)), ((), ())),
                           preferred_element_type=jnp.float32)


def _gc0_body(x_ref, w_ref, b_ref, o_ref):
    j = pl.program_id(1)
    xb = x_ref[0]
    col = lax.broadcasted_iota(jnp.int32, xb.shape, 1) + j * xb.shape[1]
    xb = jnp.where(col < N_V, xb, 0.0)
    y = _mm(w_ref[...], xb) + b_ref[...]
    colo = lax.broadcasted_iota(jnp.int32, y.shape, 1) + j * y.shape[1]
    o_ref[0] = jnp.where(colo < N_V, y, 0.0)


def _gc0(x, W, bp):
    B, CK, _ = x.shape
    CO = W.shape[0]
    NB = 256
    return pl.pallas_call(
        _gc0_body,
        grid=(B, NP // NB),
        in_specs=[
            pl.BlockSpec((1, CK, NB), lambda i, j: (i, 0, j)),
            pl.BlockSpec((CO, CK), lambda i, j: (0, 0)),
            pl.BlockSpec((CO, 1), lambda i, j: (0, 0)),
        ],
        out_specs=pl.BlockSpec((1, CO, NB), lambda i, j: (i, 0, j)),
        out_shape=jax.ShapeDtypeStruct((B, CO, NP), jnp.float32),
    )(x, W, bp)


def _sl(pv, o, n):
    return lax.slice(pv, (o, 0), (o + n, 1))


def _block_vals(h, Bm, mats, pv, d, mask):
    in_c, half = d['in_c'], d['half']
    vo = d['vo']
    w1, cw, w2 = mats[0], mats[1], mats[2]
    y = _gn_relu(h, _sl(pv, vo['pre_norm_g'], in_c),
                 _sl(pv, vo['pre_norm_b'], in_c), in_c // 8, mask)
    y1 = (_mm(w1, y) + _sl(pv, vo['lin1_b'], half)) * mask
    y1 = _gn_relu(y1, _sl(pv, vo['norm1_g'], half),
                  _sl(pv, vo['norm1_b'], half), half // 8, mask)
    s_cm = _mm_tl(cw, y1)
    z = (jnp.dot(s_cm, Bm, preferred_element_type=jnp.float32)
         + _sl(pv, vo['conv_b'], half)) * mask
    y2 = _gn_relu(z, _sl(pv, vo['norm2_g'], half),
                  _sl(pv, vo['norm2_b'], half), half // 8, mask)
    y3 = _mm(w2, y2) + _sl(pv, vo['lin2_b'], d['out_c'])
    if d['skip']:
        hs = _mm(mats[3], h) + _sl(pv, vo['skip_b'], d['out_c'])
    else:
        hs = h
    return (hs + y3) * mask


def _mega_body(descs, nm, ho, *refs):
    h0_ref, B_ref, pv_ref = refs[0], refs[1], refs[2]
    mats = [r[...] for r in refs[3:3 + nm]]
    slw, slb, cgw, cgb, clw, clb, shp_ref, cam_ref = refs[3 + nm:]
    mask = _col_mask()
    pv = pv_ref[...]
    Bm = B_ref[...]

    h = h0_ref[0]
    for d in descs[:4]:
        nm_d = 4 if d['skip'] else 3
        h = _block_vals(h, Bm, mats[d['mi']:d['mi'] + nm_d], pv, d, mask)
    s = h
    for d in descs[4:]:
        nm_d = 4 if d['skip'] else 3
        s = _block_vals(s, Bm, mats[d['mi']:d['mi'] + nm_d], pv, d, mask)

    sn = _gn_relu(s, _sl(pv, ho['sg'], 32), _sl(pv, ho['sb'], 32), 4, mask)
    shp = _mm(slw[...], sn) + slb[...]
    shp_ref[0] = lax.slice(shp, (0, 0), (3, N_V))
    hn = _gn_relu(h, _sl(pv, ho['cg'], 512), _sl(pv, ho['cb'], 512), 64, mask)
    c = jnp.maximum(_mm(cgw[...], hn) + cgb[...], 0.0) * mask
    clwp = jnp.pad(clw[...], ((0, 0), (0, NP - N_V)))
    cam = lax.dot_general(c, clwp, (((1,), (1,)), ((), ())),
                          preferred_element_type=jnp.float32)
    cam_ref[0] = jnp.pad(cam, ((0, 0), (0, 128 - 3))) + clb[...]


def _mega(h0, Bmat, params):
    B = h0.shape[0]
    blocks = list(params['gc_blocks']) + list(params['shape_blocks'])

    mats, vecs, descs = [], [], []
    voff = [0]

    def addvec(a):
        o = voff[0]
        vecs.append(a.reshape(-1, 1))
        voff[0] += a.shape[0]
        return o

    for p in blocks:
        d = {'in_c': p['lin1_W'].shape[1], 'half': p['conv_W'].shape[0],
             'out_c': p['lin2_W'].shape[0], 'skip': 'skip_W' in p,
             'mi': len(mats)}
        mats += [p['lin1_W'], p['conv_W'], p['lin2_W']]
        d['vo'] = {k: addvec(p[k]) for k in
                   ['pre_norm_g', 'pre_norm_b', 'lin1_b', 'norm1_g',
                    'norm1_b', 'conv_b', 'norm2_g', 'norm2_b', 'lin2_b']}
        if d['skip']:
            mats.append(p['skip_W'])
            d['vo']['skip_b'] = addvec(p['skip_b'])
        descs.append(d)

    ho = {'sg': addvec(params['shape_gn_g']),
          'sb': addvec(params['shape_gn_b']),
          'cg': addvec(params['cam_gn_g']),
          'cb': addvec(params['cam_gn_b'])}
    pv = jnp.concatenate(vecs, axis=0)

    slw = jnp.zeros((8, 32), jnp.float32).at[:3].set(params['shape_lin_W'])
    slb = jnp.zeros((8, 1), jnp.float32).at[:3, 0].set(params['shape_lin_b'])
    cgw = jnp.zeros((8, 512), jnp.float32).at[:1].set(params['cam_glin_W'])
    cgb = jnp.zeros((8, 1), jnp.float32).at[:1, 0].set(params['cam_glin_b'])
    clw = params['cam_lin_W']
    clb = jnp.zeros((1, 128), jnp.float32).at[0, :3].set(params['cam_lin_b'])

    ops = [h0, Bmat, pv] + mats + [slw, slb, cgw, cgb, clw, clb]
    specs = [pl.BlockSpec((1, h0.shape[1], NP), lambda i: (i, 0, 0))]
    specs += [pl.BlockSpec(a.shape, lambda i, _r=a.ndim: tuple(0 for _ in range(_r)))
              for a in ops[1:]]

    return pl.pallas_call(
        functools.partial(_mega_body, descs, len(mats), ho),
        grid=(B,),
        in_specs=specs,
        out_specs=[pl.BlockSpec((1, 3, N_V), lambda i: (i, 0, 0)),
                   pl.BlockSpec((1, 8, 128), lambda i: (i, 0, 0))],
        out_shape=[jax.ShapeDtypeStruct((B, 3, N_V), jnp.float32),
                   jax.ShapeDtypeStruct((B, 8, 128), jnp.float32)],
    )(*ops)


_HALF = 896
_EPT = 896
_NNZ_PAD = 16 * _EPT
_STRIPE = _HALF * NP // 16
_ZCH = _STRIPE // 8


def _sc_body(rows_hbm, cols_hbm, vals_hbm, out_hbm,
             sbuf, rowv, colv, valv, idx2, val2, zbuf, sem):
    cid = lax.axis_index("c")
    sid = lax.axis_index("s")
    base = sid * _EPT
    pltpu.sync_copy(rows_hbm.at[pl.ds(base, _EPT)], rowv)
    pltpu.sync_copy(cols_hbm.at[pl.ds(base, _EPT)], colv)
    pltpu.sync_copy(vals_hbm.at[pl.ds(base, _EPT)], valv)

    def zb(i, carry):
        zbuf[pl.ds(i * 16, 16)] = jnp.zeros((16,), jnp.float32)
        return carry
    lax.fori_loop(0, _ZCH // 16, zb, 0)
    for t in range(8):
        pltpu.sync_copy(zbuf, sbuf.at[pl.ds(sid * _STRIPE + t * _ZCH, _ZCH)])

    ofs = cid * _HALF
    for k in range(_EPT // 16):
        r16 = rowv[pl.ds(k * 16, 16)]
        c16 = colv[pl.ds(k * 16, 16)]
        inh = (c16 >= ofs) & (c16 < ofs + _HALF)
        flat = jnp.where(inh, (c16 - ofs) * NP + r16, N_V)
        idx2[k // 8, pl.ds((k % 8) * 16, 16)] = flat
        val2[k // 8, pl.ds((k % 8) * 16, 16)] = valv[pl.ds(k * 16, 16)]

    plsc.subcore_barrier()
    for j in range(7):
        pltpu.sync_copy(val2.at[j], sbuf.at[idx2.at[j]], add=True)
    plsc.subcore_barrier()
    r0 = cid * _HALF + sid * 56
    copies = []
    for r in range(56):
        copies.append(pltpu.async_copy(
            sbuf.at[pl.ds((sid * 56 + r) * NP, NP)],
            out_hbm.at[r0 + r], sem))
    for c in copies:
        c.wait()


def _build_B(adj_indices, adj_values):
    pad = _NNZ_PAD - adj_values.shape[0]
    rows = jnp.pad(adj_indices[0], (0, pad))
    cols = jnp.pad(adj_indices[1], (0, pad), constant_values=2 * NP)
    vals = jnp.pad(adj_values, (0, pad))

    mesh = plsc.VectorSubcoreMesh(core_axis_name="c", subcore_axis_name="s")
    run = functools.partial(
        pl.kernel,
        out_type=jax.ShapeDtypeStruct((NP, NP), jnp.float32),
        mesh=mesh,
        scratch_types=[
            pltpu.VMEM_SHARED((_HALF * NP,), jnp.float32),
            pltpu.VMEM((_EPT,), jnp.int32),
            pltpu.VMEM((_EPT,), jnp.int32),
            pltpu.VMEM((_EPT,), jnp.float32),
            pltpu.VMEM((7, 128), jnp.int32),
            pltpu.VMEM((7, 128), jnp.float32),
            pltpu.VMEM((_ZCH,), jnp.float32),
            pltpu.SemaphoreType.DMA,
        ],
    )(_sc_body)
    return run(rows, cols, vals)


def kernel(x, adj_indices, adj_values, params):
    Bmat = _build_B(adj_indices, adj_values)
    h = _gc0(x, params['gc0_W'], params['gc0_b'].reshape(-1, 1))
    shp, cam = _mega(h, Bmat, params)
    camera = cam[:, 0, :3]
    return (shp, camera)
